# Initial kernel scaffold; baseline (speedup 1.0000x reference)
import functools

import jax
import jax.numpy as jnp
from jax import lax
from jax.experimental import pallas as pl
from jax.experimental.pallas import tpu as pltpu
from jax.experimental.pallas import tpu_sc as plsc

BATCH = 16384
HIST = 50
EMBED = 32
B = BATCH * HIST
NC = 2
NS = 16
NW = NC * NS
B_PER_W = B // NW
CHUNK = 1600
NCHUNK = B_PER_W // CHUNK

_mesh = plsc.VectorSubcoreMesh(core_axis_name="c", subcore_axis_name="s")


@functools.partial(
    pl.kernel,
    mesh=_mesh,
    out_type=jax.ShapeDtypeStruct((B, EMBED), jnp.float32),
    scratch_types=[
        pltpu.VMEM((CHUNK,), jnp.int32),
        pltpu.VMEM((CHUNK, EMBED), jnp.float32),
        pltpu.SemaphoreType.DMA,
    ],
)
def _gather(idx_hbm, table_hbm, out_hbm, idx_v, rows_v, sem):
    wid = lax.axis_index("s") * NC + lax.axis_index("c")
    base = wid * B_PER_W

    def body(i, carry):
        off = base + i * CHUNK
        pltpu.sync_copy(idx_hbm.at[pl.ds(off, CHUNK)], idx_v)
        pltpu.async_copy(table_hbm.at[idx_v], rows_v, sem).wait()
        pltpu.sync_copy(rows_v, out_hbm.at[pl.ds(off, CHUNK)])
        return carry

    lax.fori_loop(0, NCHUNK, body, 0)


def kernel(object_index, table):
    idx = object_index.reshape(-1).astype(jnp.int32)
    out = _gather(idx, table)
    return out.reshape(BATCH, HIST, EMBED)


# speedup vs baseline: 1.1031x; 1.1031x over previous
import functools

import jax
import jax.numpy as jnp
from jax import lax
from jax.experimental import pallas as pl
from jax.experimental.pallas import tpu as pltpu
from jax.experimental.pallas import tpu_sc as plsc

BATCH = 16384
HIST = 50
EMBED = 32
B = BATCH * HIST
NC = 2
NS = 16
NW = NC * NS
B_PER_W = B // NW
CHUNK = 1600
NCHUNK = B_PER_W // CHUNK

_mesh = plsc.VectorSubcoreMesh(core_axis_name="c", subcore_axis_name="s")


@functools.partial(
    pl.kernel,
    mesh=_mesh,
    out_type=jax.ShapeDtypeStruct((B, EMBED), jnp.float32),
    scratch_types=[
        pltpu.VMEM((CHUNK,), jnp.int32),
        pltpu.VMEM((CHUNK, EMBED), jnp.float32),
        pltpu.SemaphoreType.DMA,
    ],
    compiler_params=pltpu.CompilerParams(use_tc_tiling_on_sc=False),
)
def _gather(idx_hbm, table_hbm, out_hbm, idx_v, rows_v, sem):
    wid = lax.axis_index("s") * NC + lax.axis_index("c")
    base = wid * B_PER_W

    def body(i, carry):
        off = base + i * CHUNK
        pltpu.sync_copy(idx_hbm.at[pl.ds(off, CHUNK)], idx_v)
        pltpu.async_copy(table_hbm.at[idx_v], rows_v, sem).wait()
        pltpu.sync_copy(rows_v, out_hbm.at[pl.ds(off, CHUNK)])
        return carry

    lax.fori_loop(0, NCHUNK, body, 0)


def kernel(object_index, table):
    idx = object_index.reshape(-1).astype(jnp.int32)
    out = _gather(idx, table)
    return out.reshape(BATCH, HIST, EMBED)


# speedup vs baseline: 1.1118x; 1.0079x over previous
import functools

import jax
import jax.numpy as jnp
from jax import lax
from jax.experimental import pallas as pl
from jax.experimental.pallas import tpu as pltpu
from jax.experimental.pallas import tpu_sc as plsc

BATCH = 16384
HIST = 50
EMBED = 32
B = BATCH * HIST
NC = 2
NS = 16
NW = NC * NS
B_PER_W = B // NW
NBUF = 4
CHUNK = 800
NGROUP = B_PER_W // (NBUF * CHUNK)

_mesh = plsc.VectorSubcoreMesh(core_axis_name="c", subcore_axis_name="s")


@functools.partial(
    pl.kernel,
    mesh=_mesh,
    out_type=jax.ShapeDtypeStruct((B, EMBED), jnp.float32),
    scratch_types=[
        pltpu.VMEM((NBUF, CHUNK), jnp.int32),
        pltpu.VMEM((NBUF, CHUNK, EMBED), jnp.float32),
    ]
    + [pltpu.SemaphoreType.DMA] * (3 * NBUF),
    compiler_params=pltpu.CompilerParams(use_tc_tiling_on_sc=False),
)
def _gather(idx_hbm, table_hbm, out_hbm, idx_v, rows_v, *sems):
    sem_i = sems[0:NBUF]
    sem_g = sems[NBUF:2 * NBUF]
    sem_o = sems[2 * NBUF:]

    wid = lax.axis_index("s") * NC + lax.axis_index("c")
    base = wid * B_PER_W

    def idx_copy(g, b):
        off = base + (g * NBUF + b) * CHUNK
        return pltpu.make_async_copy(
            idx_hbm.at[pl.ds(off, CHUNK)], idx_v.at[b], sem_i[b])

    def gather_copy(b):
        return pltpu.make_async_copy(
            table_hbm.at[idx_v.at[b]], rows_v.at[b], sem_g[b])

    def out_copy(g, b):
        off = base + (g * NBUF + b) * CHUNK
        return pltpu.make_async_copy(
            rows_v.at[b], out_hbm.at[pl.ds(off, CHUNK)], sem_o[b])

    for b in range(NBUF):
        idx_copy(0, b).start()

    def group(g, first, last):
        for b in range(NBUF):
            idx_copy(g, b).wait()
            if not first:
                out_copy(g - 1, b).wait()
            gather_copy(b).start()
        for b in range(NBUF):
            gather_copy(b).wait()
            if not last:
                idx_copy(g + 1, b).start()
            out_copy(g, b).start()

    group(0, True, NGROUP == 1)

    def body(g, carry):
        group(g, False, False)
        return carry

    lax.fori_loop(1, NGROUP - 1, body, 0)
    group(NGROUP - 1, False, True)

    for b in range(NBUF):
        out_copy(NGROUP - 1, b).wait()


def kernel(object_index, table):
    idx = object_index.reshape(-1).astype(jnp.int32)
    out = _gather(idx, table)
    return out.reshape(BATCH, HIST, EMBED)


# speedup vs baseline: 1.7492x; 1.5732x over previous
import functools

import jax
import jax.numpy as jnp
from jax import lax
from jax.experimental import pallas as pl
from jax.experimental.pallas import tpu as pltpu
from jax.experimental.pallas import tpu_sc as plsc

BATCH = 16384
HIST = 50
EMBED = 32
NC = 2
NS = 16
NW = NC * NS
NBT = BATCH // 128
NUNIT = HIST * NBT
U_PER_W = NUNIT // NW
NBUF = 4
NGROUP = U_PER_W // NBUF

_mesh = plsc.VectorSubcoreMesh(core_axis_name="c", subcore_axis_name="s")


@functools.partial(
    pl.kernel,
    mesh=_mesh,
    out_type=jax.ShapeDtypeStruct((HIST, 4, NBT, 1024), jnp.float32),
    scratch_types=[
        pltpu.VMEM((NBUF, 128), jnp.int32),
        pltpu.VMEM((NBUF, 128, EMBED), jnp.float32),
        pltpu.VMEM((NBUF, 4096), jnp.float32),
    ]
    + [pltpu.SemaphoreType.DMA] * (3 * NBUF),
    compiler_params=pltpu.CompilerParams(use_tc_tiling_on_sc=False, needs_layout_passes=False),
)
def _gather_fmt(idx_hbm, tlin_hbm, out_hbm, idx_v, rows_v, tile_v, *sems):
    sem_i = sems[0:NBUF]
    sem_g = sems[NBUF:2 * NBUF]
    sem_o = sems[2 * NBUF:]

    wid = lax.axis_index("s") * NC + lax.axis_index("c")
    u0 = wid * U_PER_W
    iota16 = lax.iota(jnp.int32, 16)
    iota128 = iota16 * 128

    def unit_hbt(u):
        uu = u0 + u
        return uu // NBT, uu % NBT

    def idx_copy(u, b):
        h, bt = unit_hbt(u)
        return pltpu.make_async_copy(
            idx_hbm.at[h, pl.ds(bt * 128, 128)], idx_v.at[b], sem_i[b])

    def gather_copy(b):
        return pltpu.make_async_copy(
            tlin_hbm.at[idx_v.at[b]], rows_v.at[b], sem_g[b])

    def out_copy(u, b, k):
        h, bt = unit_hbt(u)
        return pltpu.make_async_copy(
            tile_v.at[b, pl.ds(k * 1024, 1024)], out_hbm.at[h, k, bt], sem_o[b])

    def transpose(b):
        def body(i, carry):
            for j in range(8):
                l = i * 8 + j
                lvec = iota16 * 0 + l
                for d0 in (0, 16):
                    v = plsc.load_gather(rows_v.at[b], [lvec, iota16 + d0])
                    plsc.store_scatter(
                        tile_v.at[b], [iota128 + (d0 * 128 + l)], v)
            return carry
        lax.fori_loop(0, 16, body, 0)

    for b in range(NBUF):
        idx_copy(b, b).start()

    def group(g, first, last):
        for b in range(NBUF):
            u = g * NBUF + b
            idx_copy(u, b).wait()
            if not first:
                for k in range(4):
                    out_copy(u - NBUF, b, k).wait()
            gather_copy(b).start()
        for b in range(NBUF):
            u = g * NBUF + b
            gather_copy(b).wait()
            if not last:
                idx_copy(u + NBUF, b).start()
            transpose(b)
            for k in range(4):
                out_copy(u, b, k).start()

    group(0, True, NGROUP == 1)

    def body(g, carry):
        group(g, False, False)
        return carry

    lax.fori_loop(1, NGROUP - 1, body, 0)
    group(NGROUP - 1, False, True)

    for b in range(NBUF):
        for k in range(4):
            out_copy((NGROUP - 1) * NBUF + b, b, k).wait()


def kernel(object_index, table):
    idx_t = object_index.astype(jnp.int32).T
    out = _gather_fmt(idx_t, table)
    out5 = out.reshape(HIST, 4, NBT, 8, 128)
    return out5.transpose(2, 4, 0, 1, 3).reshape(BATCH, HIST, EMBED)
